# baseline (speedup 1.0000x reference)
import functools

import jax
import jax.numpy as jnp
from jax import lax
from jax.experimental import pallas as pl
from jax.experimental.pallas import tpu as pltpu
from jax.experimental.pallas import tpu_sc as plsc

_NC = 2
_NS = 16
_LANES = 16
_CHUNK_ROWS = 8


def _w_table_kernel(vec_ref, attr_t_ref, out_ref):
    out_ref[...] = lax.dot_general(
        vec_ref[...], attr_t_ref[...],
        (((1,), (0,)), ((), ())),
        preferred_element_type=jnp.float32,
    )


def _make_sc_gather(n, e, l):
    mesh = plsc.VectorSubcoreMesh(core_axis_name="c", subcore_axis_name="s")
    nw = _NC * _NS
    rows_per_w = n // nw
    n_chunks = rows_per_w // _CHUNK_ROWS
    col_groups = n // _LANES
    groups = _CHUNK_ROWS * col_groups

    @functools.partial(
        pl.kernel,
        out_type=jax.ShapeDtypeStruct((n, n), jnp.float32),
        mesh=mesh,
        compiler_params=pltpu.CompilerParams(
            needs_layout_passes=False, use_tc_tiling_on_sc=True),
        scratch_types=[
            [pltpu.VMEM((1, e), jnp.float32) for _ in range(l)],
            [[pltpu.VMEM((_CHUNK_ROWS, n), jnp.int32) for _ in range(l)]
             for _ in range(2)],
            pltpu.VMEM((_CHUNK_ROWS, n), jnp.float32),
            pltpu.SemaphoreType.DMA,
            pltpu.SemaphoreType.DMA,
            pltpu.SemaphoreType.DMA,
        ],
    )
    def sc_gather(w_hbm, paths_hbm, out_hbm, tables_v, idx_v, out_v,
                  sem_t, sem_i, sem_s):
        wid = lax.axis_index("s") * _NC + lax.axis_index("c")
        row0 = wid * rows_per_w
        in_dmas = [
            pltpu.async_copy(w_hbm.at[pl.ds(i, 1), :], tables_v[i], sem_t)
            for i in range(l)
        ]
        idx_dmas = [
            [pltpu.async_copy(
                paths_hbm.at[c, pl.ds(row0 + ch * _CHUNK_ROWS, _CHUNK_ROWS), :],
                idx_v[ch][c], sem_i)
             for c in range(l)]
            for ch in range(n_chunks)
        ]
        for dma in in_dmas:
            dma.wait()

        scale = jnp.float32(1.0 / l)
        store_dma = None
        for ch in range(n_chunks):
            for dma in idx_dmas[ch]:
                dma.wait()
            if store_dma is not None:
                store_dma.wait()
            idx_ch = idx_v[ch]

            @plsc.parallel_loop(0, groups, unroll=2)
            def _grp(g):
                r8 = g // col_groups
                c0 = (g % col_groups) * _LANES
                acc = jnp.zeros((_LANES,), jnp.float32)
                zero = jnp.zeros((_LANES,), jnp.int32)
                for i in range(l):
                    ev = idx_ch[i][r8, pl.ds(c0, _LANES)]
                    acc = acc + plsc.load_gather(tables_v[i], [zero, ev])
                out_v[r8, pl.ds(c0, _LANES)] = acc * scale

            store_dma = pltpu.async_copy(
                out_v,
                out_hbm.at[pl.ds(row0 + ch * _CHUNK_ROWS, _CHUNK_ROWS), :],
                sem_s)
        store_dma.wait()

    return sc_gather


def kernel(x, edge_attr, edge_paths, edge_vector):
    n = edge_paths.shape[0]
    l, d_edge = edge_vector.shape
    e = edge_attr.shape[0]

    w = pl.pallas_call(
        _w_table_kernel,
        out_shape=jax.ShapeDtypeStruct((l, e), jnp.float32),
    )(edge_vector, edge_attr.T)

    paths_hm = edge_paths.transpose(2, 0, 1)
    sc_gather = _make_sc_gather(n, e, l)
    return sc_gather(w, paths_hm)


# speedup vs baseline: 1.0621x; 1.0621x over previous
import functools

import jax
import jax.numpy as jnp
from jax import lax
from jax.experimental import pallas as pl
from jax.experimental.pallas import tpu as pltpu
from jax.experimental.pallas import tpu_sc as plsc

_NC = 2
_NS = 16
_LANES = 16
_CHUNK_ROWS = 8


def _w_table_kernel(vec_ref, attr_t_ref, out_ref):
    out_ref[...] = lax.dot_general(
        vec_ref[...], attr_t_ref[...],
        (((1,), (0,)), ((), ())),
        preferred_element_type=jnp.float32,
    )


def _make_sc_gather(n, e, l):
    mesh = plsc.VectorSubcoreMesh(core_axis_name="c", subcore_axis_name="s", num_cores=1)
    nw = 1 * _NS
    rows_per_w = n // nw
    n_chunks = rows_per_w // _CHUNK_ROWS
    col_groups = n // _LANES
    groups = _CHUNK_ROWS * col_groups

    @functools.partial(
        pl.kernel,
        out_type=jax.ShapeDtypeStruct((n, n), jnp.float32),
        mesh=mesh,
        compiler_params=pltpu.CompilerParams(
            needs_layout_passes=False, use_tc_tiling_on_sc=True),
        scratch_types=[
            [pltpu.VMEM((1, e), jnp.float32) for _ in range(l)],
            [[pltpu.VMEM((_CHUNK_ROWS, n), jnp.int32) for _ in range(l)]
             for _ in range(2)],
            pltpu.VMEM((_CHUNK_ROWS, n), jnp.float32),
            pltpu.SemaphoreType.DMA,
            pltpu.SemaphoreType.DMA,
            pltpu.SemaphoreType.DMA,
        ],
    )
    def sc_gather(w_hbm, paths_hbm, out_hbm, tables_v, idx_v, out_v,
                  sem_t, sem_i, sem_s):
        wid = lax.axis_index("s")
        row0 = wid * rows_per_w
        in_dmas = [
            pltpu.async_copy(w_hbm.at[pl.ds(i, 1), :], tables_v[i], sem_t)
            for i in range(l)
        ]
        def issue_idx(ch):
            buf = ch % 2
            return [pltpu.async_copy(
                paths_hbm.at[c, pl.ds(row0 + ch * _CHUNK_ROWS, _CHUNK_ROWS), :],
                idx_v[buf][c], sem_i)
                for c in range(l)]

        idx_dmas = [issue_idx(0), issue_idx(1)]
        for dma in in_dmas:
            dma.wait()

        scale = jnp.float32(1.0 / l)
        store_dma = None
        for ch in range(n_chunks):
            buf = ch % 2
            for dma in idx_dmas[buf]:
                dma.wait()
            if store_dma is not None:
                store_dma.wait()
            idx_ch = idx_v[buf]

            @plsc.parallel_loop(0, groups, unroll=2)
            def _grp(g):
                r8 = g // col_groups
                c0 = (g % col_groups) * _LANES
                acc = jnp.zeros((_LANES,), jnp.float32)
                zero = jnp.zeros((_LANES,), jnp.int32)
                for i in range(l):
                    ev = idx_ch[i][r8, pl.ds(c0, _LANES)]
                    acc = acc + plsc.load_gather(tables_v[i], [zero, ev])
                out_v[r8, pl.ds(c0, _LANES)] = acc * scale

            if ch + 2 < n_chunks:
                idx_dmas[buf] = issue_idx(ch + 2)
            store_dma = pltpu.async_copy(
                out_v,
                out_hbm.at[pl.ds(row0 + ch * _CHUNK_ROWS, _CHUNK_ROWS), :],
                sem_s)
        store_dma.wait()

    return sc_gather


def kernel(x, edge_attr, edge_paths, edge_vector):
    n = edge_paths.shape[0]
    l, d_edge = edge_vector.shape
    e = edge_attr.shape[0]

    w = pl.pallas_call(
        _w_table_kernel,
        out_shape=jax.ShapeDtypeStruct((l, e), jnp.float32),
    )(edge_vector, edge_attr.T)

    paths_hm = edge_paths.transpose(2, 0, 1)
    sc_gather = _make_sc_gather(n, e, l)
    return sc_gather(w, paths_hm)
